# Initial kernel scaffold; baseline (speedup 1.0000x reference)
import jax
import jax.numpy as jnp
from jax.experimental import pallas as pl


def kernel(x, Wg, We):
    raise NotImplementedError("write your pallas kernel here")


# speedup vs baseline: 4.7937x; 4.7937x over previous
import functools

import jax
import jax.numpy as jnp
from jax.experimental import pallas as pl
from jax.experimental.pallas import tpu as pltpu

NUM_EXPERTS = 64
GROUP_SIZE = 4
NUM_GROUPS = 16
TOP_K = 2
LANES = 128


def _router_body(n_tok, x_ref, w_ref, fw_ref, mask_ref, loss_ref,
                 load_acc, zg_acc, zl_acc):
    i = pl.program_id(0)
    n_steps = pl.num_programs(0)

    logits = jnp.dot(x_ref[...], w_ref[...],
                     preferred_element_type=jnp.float32,
                     precision=jax.lax.Precision.DEFAULT)
    t = logits.shape[0]
    lane = jax.lax.broadcasted_iota(jnp.int32, (t, LANES), 1).astype(jnp.float32)
    is_g = lane < float(NUM_GROUPS)
    is_l = (lane >= float(NUM_GROUPS)) & (lane < float(NUM_GROUPS + GROUP_SIZE))

    gval = jnp.where(is_g, logits, -1e30)
    m_g = jnp.max(gval, axis=1, keepdims=True)
    s_g = jnp.sum(jnp.where(is_g, jnp.exp(logits - m_g), 0.0),
                  axis=1, keepdims=True)
    cg = jnp.min(jnp.where(gval >= m_g, lane, 1e9), axis=1, keepdims=True)
    cgw = 1.0 / s_g

    lval = jnp.where(is_l, logits, -1e30)
    m1 = jnp.max(lval, axis=1, keepdims=True)
    i1 = jnp.min(jnp.where(lval >= m1, lane, 1e9), axis=1, keepdims=True)
    lval2 = jnp.where(lane == i1, -1e30, lval)
    m2 = jnp.max(lval2, axis=1, keepdims=True)
    i2 = jnp.min(jnp.where(lval2 >= m2, lane, 1e9), axis=1, keepdims=True)
    s_l = jnp.sum(jnp.where(is_l, jnp.exp(logits - m1), 0.0),
                  axis=1, keepdims=True)
    p1 = 1.0 / s_l
    p2 = jnp.exp(m2 - m1) / s_l
    inv = 1.0 / (p1 + p2 + 1e-7)
    w1 = cgw * (p1 * inv)
    w2 = cgw * (p2 * inv)

    e1 = cg * float(GROUP_SIZE) + (i1 - float(NUM_GROUPS))
    e2 = cg * float(GROUP_SIZE) + (i2 - float(NUM_GROUPS))
    lane64 = jax.lax.broadcasted_iota(jnp.int32, (t, NUM_EXPERTS), 1).astype(jnp.float32)
    mask = jnp.where(lane64 == e1, w1, 0.0) + jnp.where(lane64 == e2, w2, 0.0)
    mask_ref[...] = mask
    lane2 = jax.lax.broadcasted_iota(jnp.int32, (t, TOP_K), 1).astype(jnp.float32)
    fw_ref[...] = jnp.where(lane2 == 0.0, w1, w2)

    @pl.when(i == 0)
    def _init():
        load_acc[...] = jnp.zeros_like(load_acc)
        zg_acc[...] = jnp.zeros_like(zg_acc)
        zl_acc[...] = jnp.zeros_like(zl_acc)

    sq = logits * logits
    load_acc[...] += jnp.sum(mask, axis=0, keepdims=True)
    zg_acc[...] += jnp.sum(jnp.where(is_g, sq, 0.0), axis=(0, 1), keepdims=True)
    zl_acc[...] += jnp.sum(jnp.where(is_l, sq, 0.0), axis=(0, 1), keepdims=True)

    @pl.when(i == n_steps - 1)
    def _fin():
        load = load_acc[...]
        target = jnp.sum(load, keepdims=True) / float(NUM_EXPERTS)
        lbl = jnp.sum((load - target) ** 2, keepdims=True) / float(NUM_EXPERTS)
        z = (zg_acc[...] / float(n_tok * NUM_GROUPS)
             + zl_acc[...] / float(n_tok * GROUP_SIZE))
        loss_ref[...] = 0.001 * (lbl + z)


@jax.jit
def kernel(x, Wg, We):
    b, s, d = x.shape
    x_flat = x.reshape(-1, d)
    n_tok = x_flat.shape[0]
    w = jnp.zeros((d, LANES), jnp.float32)
    w = w.at[:, :NUM_GROUPS].set(Wg.T).at[:, NUM_GROUPS:NUM_GROUPS + GROUP_SIZE].set(We.T)

    tile = 1024
    grid = (n_tok // tile,)
    fw, mask, loss = pl.pallas_call(
        functools.partial(_router_body, n_tok),
        grid=grid,
        in_specs=[
            pl.BlockSpec((tile, d), lambda i: (i, 0)),
            pl.BlockSpec((d, LANES), lambda i: (0, 0)),
        ],
        out_specs=[
            pl.BlockSpec((tile, TOP_K), lambda i: (i, 0)),
            pl.BlockSpec((tile, NUM_EXPERTS), lambda i: (i, 0)),
            pl.BlockSpec((1, 1), lambda i: (0, 0)),
        ],
        out_shape=[
            jax.ShapeDtypeStruct((n_tok, TOP_K), jnp.float32),
            jax.ShapeDtypeStruct((n_tok, NUM_EXPERTS), jnp.float32),
            jax.ShapeDtypeStruct((1, 1), jnp.float32),
        ],
        scratch_shapes=[
            pltpu.VMEM((1, NUM_EXPERTS), jnp.float32),
            pltpu.VMEM((1, 1), jnp.float32),
            pltpu.VMEM((1, 1), jnp.float32),
        ],
    )(x_flat, w)
    return (fw, mask, loss[0, 0])


# speedup vs baseline: 4.8599x; 1.0138x over previous
import functools

import jax
import jax.numpy as jnp
from jax.experimental import pallas as pl
from jax.experimental.pallas import tpu as pltpu

NUM_EXPERTS = 64
GROUP_SIZE = 4
NUM_GROUPS = 16
TOP_K = 2
LANES = 128


def _router_body(n_tok, x_ref, w_ref, fw_ref, mask_ref, loss_ref,
                 load_acc, zg_acc, zl_acc):
    i = pl.program_id(0)
    n_steps = pl.num_programs(0)

    logits = jnp.dot(x_ref[...], w_ref[...],
                     preferred_element_type=jnp.float32,
                     precision=jax.lax.Precision.DEFAULT)
    t = logits.shape[0]
    gl = logits[:, :NUM_GROUPS]
    ll = logits[:, NUM_GROUPS:NUM_GROUPS + GROUP_SIZE]

    lane_g = jax.lax.broadcasted_iota(jnp.int32, (t, NUM_GROUPS), 1).astype(jnp.float32)
    m_g = jnp.max(gl, axis=1, keepdims=True)
    s_g = jnp.sum(jnp.exp(gl - m_g), axis=1, keepdims=True)
    cg = jnp.min(jnp.where(gl >= m_g, lane_g, 1e9), axis=1, keepdims=True)
    cgw = 1.0 / s_g

    lane_l = jax.lax.broadcasted_iota(jnp.int32, (t, GROUP_SIZE), 1).astype(jnp.float32)
    m1 = jnp.max(ll, axis=1, keepdims=True)
    i1 = jnp.min(jnp.where(ll >= m1, lane_l, 1e9), axis=1, keepdims=True)
    lval2 = jnp.where(lane_l == i1, -1e30, ll)
    m2 = jnp.max(lval2, axis=1, keepdims=True)
    i2 = jnp.min(jnp.where(lval2 >= m2, lane_l, 1e9), axis=1, keepdims=True)
    s_l = jnp.sum(jnp.exp(ll - m1), axis=1, keepdims=True)
    p1 = 1.0 / s_l
    p2 = jnp.exp(m2 - m1) / s_l
    inv = 1.0 / (p1 + p2 + 1e-7)
    w1 = cgw * (p1 * inv)
    w2 = cgw * (p2 * inv)

    e1 = cg * float(GROUP_SIZE) + i1
    e2 = cg * float(GROUP_SIZE) + i2
    lane64 = jax.lax.broadcasted_iota(jnp.int32, (t, NUM_EXPERTS), 1).astype(jnp.float32)
    mask = jnp.where(lane64 == e1, w1, 0.0) + jnp.where(lane64 == e2, w2, 0.0)
    mask_ref[...] = mask
    lane2 = jax.lax.broadcasted_iota(jnp.int32, (t, TOP_K), 1).astype(jnp.float32)
    fw_ref[...] = jnp.where(lane2 == 0.0, w1, w2)

    @pl.when(i == 0)
    def _init():
        load_acc[...] = jnp.zeros_like(load_acc)
        zg_acc[...] = jnp.zeros_like(zg_acc)
        zl_acc[...] = jnp.zeros_like(zl_acc)

    load_acc[...] += jnp.sum(mask, axis=0, keepdims=True)
    zg_acc[...] += jnp.sum(gl * gl, axis=(0, 1), keepdims=True)
    zl_acc[...] += jnp.sum(ll * ll, axis=(0, 1), keepdims=True)

    @pl.when(i == n_steps - 1)
    def _fin():
        load = load_acc[...]
        target = jnp.sum(load, keepdims=True) / float(NUM_EXPERTS)
        lbl = jnp.sum((load - target) ** 2, keepdims=True) / float(NUM_EXPERTS)
        z = (zg_acc[...] / float(n_tok * NUM_GROUPS)
             + zl_acc[...] / float(n_tok * GROUP_SIZE))
        loss_ref[...] = 0.001 * (lbl + z)


@jax.jit
def kernel(x, Wg, We):
    b, s, d = x.shape
    x_flat = x.reshape(-1, d)
    n_tok = x_flat.shape[0]
    w = jnp.zeros((d, LANES), jnp.float32)
    w = w.at[:, :NUM_GROUPS].set(Wg.T).at[:, NUM_GROUPS:NUM_GROUPS + GROUP_SIZE].set(We.T)

    tile = 2048
    grid = (n_tok // tile,)
    fw, mask, loss = pl.pallas_call(
        functools.partial(_router_body, n_tok),
        grid=grid,
        in_specs=[
            pl.BlockSpec((tile, d), lambda i: (i, 0)),
            pl.BlockSpec((d, LANES), lambda i: (0, 0)),
        ],
        out_specs=[
            pl.BlockSpec((tile, TOP_K), lambda i: (i, 0)),
            pl.BlockSpec((tile, NUM_EXPERTS), lambda i: (i, 0)),
            pl.BlockSpec((1, 1), lambda i: (0, 0)),
        ],
        out_shape=[
            jax.ShapeDtypeStruct((n_tok, TOP_K), jnp.float32),
            jax.ShapeDtypeStruct((n_tok, NUM_EXPERTS), jnp.float32),
            jax.ShapeDtypeStruct((1, 1), jnp.float32),
        ],
        scratch_shapes=[
            pltpu.VMEM((1, NUM_EXPERTS), jnp.float32),
            pltpu.VMEM((1, 1), jnp.float32),
            pltpu.VMEM((1, 1), jnp.float32),
        ],
    )(x_flat, w)
    return (fw, mask, loss[0, 0])
